# baseline (speedup 1.0000x reference)
import functools

import jax
import jax.numpy as jnp
from jax import lax
from jax.experimental import pallas as pl
from jax.experimental.pallas import tpu as pltpu
from jax.experimental.pallas import tpu_sc as plsc

N = 10000
E = 160000
D = 128

NC, NS = 2, 16
NW = NC * NS
E4 = 163840
CHUNK = 128
NSLICE = 4
EH = E4 // NSLICE
NCH = EH // CHUNK
TW = NCH // NW
NBUF = 4
GDQ = 2
GDW = 3


def _proj_body(feat_ref, w1s_ref, w1d_ref, wfb_ref, p2_ref, q_ref):
    f = feat_ref[...]
    p = jnp.dot(f, w1s_ref[...], preferred_element_type=jnp.float32)
    p2_ref[0] = p + wfb_ref[0:1, :]
    p2_ref[1] = p + wfb_ref[1:2, :]
    q_ref[...] = jnp.dot(f, w1d_ref[...], preferred_element_type=jnp.float32)


def _node_projections(feat, w1s, w1d, wfb):
    bn = 2000
    grid = (N // bn,)
    p2, q = pl.pallas_call(
        _proj_body,
        grid=grid,
        in_specs=[
            pl.BlockSpec((bn, D), lambda i: (i, 0)),
            pl.BlockSpec((D, D), lambda i: (0, 0)),
            pl.BlockSpec((D, D), lambda i: (0, 0)),
            pl.BlockSpec((2, D), lambda i: (0, 0)),
        ],
        out_specs=[
            pl.BlockSpec((2, bn, D), lambda i: (0, i, 0)),
            pl.BlockSpec((bn, D), lambda i: (i, 0)),
        ],
        out_shape=[
            jax.ShapeDtypeStruct((2, N, D), jnp.float32),
            jax.ShapeDtypeStruct((N, D), jnp.float32),
        ],
    )(feat, w1s, w1d, wfb)
    return p2.reshape(2 * N, D), q


def _sc_gather_body(p2_hbm, q_hbm, src_hbm, dst_hbm, g_hbm,
                    src_v, dst_v, sx_v, buf, bufb, semI, semP, semQ, semW):
    cid = lax.axis_index("c")
    sid = lax.axis_index("s")
    ch0 = (cid * NS + sid) * TW
    T = TW

    def fire_idx(c):
        b = lax.rem(c, NBUF)
        pltpu.async_copy(src_hbm.at[ch0 + c], src_v.at[b], semI.at[b])
        pltpu.async_copy(dst_hbm.at[ch0 + c], dst_v.at[b], semI.at[b])

    fire_idx(0)

    def body(t, _):
        b = lax.rem(t, NBUF)

        @pl.when(t < T)
        def _():
            @pl.when(t >= NBUF)
            def _():
                pltpu.make_async_copy(bufb.at[pl.ds(0, 128)],
                                      g_hbm.at[pl.ds(0, 128)],
                                      semW.at[b]).wait()
            pltpu.make_async_copy(src_hbm.at[0], src_v.at[b], semI.at[b]).wait()
            pltpu.make_async_copy(dst_hbm.at[0], dst_v.at[b], semI.at[b]).wait()
            for j in range(8):
                s = src_v[b, pl.ds(j * 16, 16)]
                d = dst_v[b, pl.ds(j * 16, 16)]
                sx_v[b, pl.ds(j * 16, 16)] = jnp.where(s == d, s + N, s)
            pltpu.async_copy(p2_hbm.at[sx_v.at[b]],
                             buf.at[pl.ds(b * 128, 128)], semP.at[b])

        cq = t - GDQ

        @pl.when((cq >= 0) & (cq < T))
        def _():
            bq = lax.rem(cq, NBUF)
            pltpu.make_async_copy(p2_hbm.at[pl.ds(0, 128)],
                                  buf.at[pl.ds(bq * 128, 128)],
                                  semP.at[bq]).wait()
            pltpu.async_copy(q_hbm.at[dst_v.at[bq]],
                             buf.at[pl.ds(bq * 128, 128)],
                             semQ.at[bq], add=True)

        cw = t - GDW

        @pl.when((cw >= 0) & (cw < T))
        def _():
            bw = lax.rem(cw, NBUF)
            bb = lax.rem(cw, 2)
            pltpu.make_async_copy(p2_hbm.at[pl.ds(0, 128)],
                                  buf.at[pl.ds(bw * 128, 128)],
                                  semQ.at[bw]).wait()

            def pack_row(r, _):
                row = bw * 128 + r
                brow = bb * 128 + r
                for j in range(4):
                    a = buf[row, pl.ds(32 * j, 16)]
                    b = buf[row, pl.ds(32 * j + 16, 16)]
                    w = plsc.bitcast(
                        plsc.pack(a, b, format=plsc.PackFormat.INTERLEAVED),
                        jnp.int32)
                    bufb[brow, pl.ds(16 * j, 16)] = w
                return 0

            lax.fori_loop(0, 128, pack_row, 0)
            pltpu.async_copy(bufb.at[pl.ds(bb * 128, 128)],
                             g_hbm.at[pl.ds((ch0 + cw) * 128, 128)],
                             semW.at[bw])

        @pl.when(t + 1 < T)
        def _():
            fire_idx(t + 1)

        return 0

    lax.fori_loop(0, T + GDW, body, 0)

    for k in range(min(NBUF, TW)):
        bk = (TW - 1 - k) % NBUF
        pltpu.make_async_copy(bufb.at[pl.ds(0, 128)],
                              g_hbm.at[pl.ds(0, 128)],
                              semW.at[bk]).wait()


def _sc_gather(p2, q, src2d, dst2d):
    mesh = plsc.VectorSubcoreMesh(
        core_axis_name="c", subcore_axis_name="s",
        num_cores=NC, num_subcores=NS)
    fn = pl.kernel(
        _sc_gather_body,
        out_type=jax.ShapeDtypeStruct((EH, D // 2), jnp.int32),
        mesh=mesh,
        compiler_params=pltpu.CompilerParams(needs_layout_passes=False),
        scratch_types=[
            pltpu.VMEM((NBUF, 128), jnp.int32),
            pltpu.VMEM((NBUF, 128), jnp.int32),
            pltpu.VMEM((NBUF, 128), jnp.int32),
            pltpu.VMEM((NBUF * 128, D), jnp.float32),
            pltpu.VMEM((2 * 128, D // 2), jnp.int32),
            pltpu.SemaphoreType.DMA((NBUF,)),
            pltpu.SemaphoreType.DMA((NBUF,)),
            pltpu.SemaphoreType.DMA((NBUF,)),
            pltpu.SemaphoreType.DMA((NBUF,)),
        ],
    )
    return fn(p2, q, src2d, dst2d)


BE = 1280
NB_A = EH // BE
NB_B = (E - EH) // BE


def _silu_heads(g32, wcat, b2):
    a = lax.bitcast_convert_type(g32 << 16, jnp.float32)
    b = lax.bitcast_convert_type(g32 & jnp.int32(-65536), jnp.float32)
    g = jnp.concatenate([a, b], axis=1)
    h = g * (1.0 / (1.0 + jnp.exp(-g)))
    o = jnp.dot(h, wcat, preferred_element_type=jnp.float32) + b2
    return o[:, :D], jnp.exp(o[:, D:])


def _head_body_a(g_ref, wcat_ref, b2_ref, loc_ref, scale_ref):
    loc, scale = _silu_heads(g_ref[...], wcat_ref[...], b2_ref[...])
    loc_ref[...] = loc
    scale_ref[...] = scale


def _head_body_b(g_ref, wcat_ref, b2_ref, li_ref, si_ref, loc_ref, scale_ref):
    del li_ref, si_ref
    loc, scale = _silu_heads(g_ref[...], wcat_ref[...], b2_ref[...])
    loc_ref[...] = loc
    scale_ref[...] = scale


_OUT_SHAPE = [
    jax.ShapeDtypeStruct((E, D), jnp.float32),
    jax.ShapeDtypeStruct((E, D), jnp.float32),
]
_WSPECS = [
    pl.BlockSpec((D, 2 * D), lambda i: (0, 0)),
    pl.BlockSpec((1, 2 * D), lambda i: (0, 0)),
]


def _heads_first(g, wcat, b2, nblocks):
    return pl.pallas_call(
        _head_body_a,
        grid=(nblocks,),
        in_specs=[pl.BlockSpec((BE, D // 2), lambda i: (i, 0))] + _WSPECS,
        out_specs=[
            pl.BlockSpec((BE, D), lambda i: (i, 0)),
            pl.BlockSpec((BE, D), lambda i: (i, 0)),
        ],
        out_shape=_OUT_SHAPE,
        compiler_params=pltpu.CompilerParams(
            dimension_semantics=("arbitrary",)),
    )(g, wcat, b2)


def _heads_next(g, wcat, b2, loc_init, scale_init, off, nblocks):
    return pl.pallas_call(
        _head_body_b,
        grid=(nblocks,),
        in_specs=[pl.BlockSpec((BE, D // 2), lambda i: (i, 0))] + _WSPECS + [
            pl.BlockSpec(memory_space=pl.ANY),
            pl.BlockSpec(memory_space=pl.ANY),
        ],
        out_specs=[
            pl.BlockSpec((BE, D), lambda i, off=off: (i + off, 0)),
            pl.BlockSpec((BE, D), lambda i, off=off: (i + off, 0)),
        ],
        out_shape=_OUT_SHAPE,
        input_output_aliases={3: 0, 4: 1},
        compiler_params=pltpu.CompilerParams(
            dimension_semantics=("arbitrary",)),
    )(g, wcat, b2, loc_init, scale_init)


def kernel(feat, edge_index, W1, b1, W_loc, b_loc, W_ls, b_ls):
    src = edge_index[0].astype(jnp.int32)
    dst = edge_index[1].astype(jnp.int32)
    pad_s = jnp.arange(E4 - E, dtype=jnp.int32)
    src2d = jnp.concatenate([src, pad_s]).reshape(E4 // 128, 128)
    dst2d = jnp.concatenate([dst, pad_s + 1]).reshape(E4 // 128, 128)

    w1s = W1[:D]
    w1d = W1[D:2 * D]
    wfb = jnp.stack([b1, b1 + W1[2 * D]])

    p2, q = _node_projections(feat, w1s, w1d, wfb)

    gs = [_sc_gather(p2, q, src2d[i * NCH:(i + 1) * NCH],
                     dst2d[i * NCH:(i + 1) * NCH]) for i in range(NSLICE)]

    wcat = jnp.concatenate([W_loc, W_ls], axis=1)
    perm = [32 * (c // 16) + (c % 16) for c in range(64)] \
        + [32 * (c // 16) + 16 + (c % 16) for c in range(64)]
    wcat = wcat[jnp.array(perm, dtype=jnp.int32), :]
    b2 = jnp.concatenate([b_loc, b_ls]).reshape(1, 2 * D)

    blocks_per_slice = EH // BE
    loc, scale = _heads_first(gs[0], wcat, b2, blocks_per_slice)
    for i in range(1, NSLICE):
        off = i * blocks_per_slice
        nb = min(blocks_per_slice, E // BE - off)
        loc, scale = _heads_next(gs[i], wcat, b2, loc, scale, off, nb)
    return (loc, scale)


# speedup vs baseline: 1.0494x; 1.0494x over previous
import functools

import jax
import jax.numpy as jnp
from jax import lax
from jax.experimental import pallas as pl
from jax.experimental.pallas import tpu as pltpu
from jax.experimental.pallas import tpu_sc as plsc

N = 10000
E = 160000
D = 128

NC, NS = 2, 16
NW = NC * NS
E4 = 163840
CHUNK = 128
NSLICE = 4
EH = E4 // NSLICE
NCH = EH // CHUNK
TW = NCH // NW
NBUF = 7
GDQ = 2
GDW = 4


def _proj_body(feat_ref, w1s_ref, w1d_ref, wfb_ref, p2_ref, q_ref):
    f = feat_ref[...]
    p = jnp.dot(f, w1s_ref[...], preferred_element_type=jnp.float32)
    p2_ref[0] = p + wfb_ref[0:1, :]
    p2_ref[1] = p + wfb_ref[1:2, :]
    q_ref[...] = jnp.dot(f, w1d_ref[...], preferred_element_type=jnp.float32)


def _node_projections(feat, w1s, w1d, wfb):
    bn = 2000
    grid = (N // bn,)
    p2, q = pl.pallas_call(
        _proj_body,
        grid=grid,
        in_specs=[
            pl.BlockSpec((bn, D), lambda i: (i, 0)),
            pl.BlockSpec((D, D), lambda i: (0, 0)),
            pl.BlockSpec((D, D), lambda i: (0, 0)),
            pl.BlockSpec((2, D), lambda i: (0, 0)),
        ],
        out_specs=[
            pl.BlockSpec((2, bn, D), lambda i: (0, i, 0)),
            pl.BlockSpec((bn, D), lambda i: (i, 0)),
        ],
        out_shape=[
            jax.ShapeDtypeStruct((2, N, D), jnp.float32),
            jax.ShapeDtypeStruct((N, D), jnp.float32),
        ],
    )(feat, w1s, w1d, wfb)
    return p2.reshape(2 * N, D), q


def _sc_gather_body(p2_hbm, q_hbm, src_hbm, dst_hbm, g_hbm,
                    src_v, dst_v, sx_v, buf, semI, semP, semQ, semW):
    cid = lax.axis_index("c")
    sid = lax.axis_index("s")
    ch0 = (cid * NS + sid) * TW
    T = TW

    def fire_idx(c):
        b = lax.rem(c, NBUF)
        pltpu.async_copy(src_hbm.at[ch0 + c], src_v.at[b], semI.at[b])
        pltpu.async_copy(dst_hbm.at[ch0 + c], dst_v.at[b], semI.at[b])

    fire_idx(0)

    def body(t, _):
        b = lax.rem(t, NBUF)

        @pl.when(t < T)
        def _():
            @pl.when(t >= NBUF)
            def _():
                pltpu.make_async_copy(buf.at[pl.ds(b * 128, 128)],
                                      g_hbm.at[pl.ds(0, 128)],
                                      semW.at[b]).wait()
            pltpu.make_async_copy(src_hbm.at[0], src_v.at[b], semI.at[b]).wait()
            pltpu.make_async_copy(dst_hbm.at[0], dst_v.at[b], semI.at[b]).wait()
            for j in range(8):
                s = src_v[b, pl.ds(j * 16, 16)]
                d = dst_v[b, pl.ds(j * 16, 16)]
                sx_v[b, pl.ds(j * 16, 16)] = jnp.where(s == d, s + N, s)
            pltpu.async_copy(p2_hbm.at[sx_v.at[b]],
                             buf.at[pl.ds(b * 128, 128)], semP.at[b])

        cq = t - GDQ

        @pl.when((cq >= 0) & (cq < T))
        def _():
            bq = lax.rem(cq, NBUF)
            pltpu.make_async_copy(g_hbm.at[pl.ds(0, 128)],
                                  buf.at[pl.ds(bq * 128, 128)],
                                  semP.at[bq]).wait()
            pltpu.async_copy(q_hbm.at[dst_v.at[bq]],
                             buf.at[pl.ds(bq * 128, 128)],
                             semQ.at[bq], add=True)

        cw = t - GDW

        @pl.when((cw >= 0) & (cw < T))
        def _():
            bw = lax.rem(cw, NBUF)
            pltpu.make_async_copy(g_hbm.at[pl.ds(0, 128)],
                                  buf.at[pl.ds(bw * 128, 128)],
                                  semQ.at[bw]).wait()
            pltpu.async_copy(buf.at[pl.ds(bw * 128, 128)],
                             g_hbm.at[pl.ds((ch0 + cw) * 128, 128)],
                             semW.at[bw])

        @pl.when(t + 1 < T)
        def _():
            fire_idx(t + 1)

        return 0

    lax.fori_loop(0, T + GDW, body, 0)

    for k in range(min(NBUF, TW)):
        bk = (TW - 1 - k) % NBUF
        pltpu.make_async_copy(buf.at[pl.ds(bk * 128, 128)],
                              g_hbm.at[pl.ds(0, 128)],
                              semW.at[bk]).wait()


def _sc_gather(p2, q, src2d, dst2d):
    mesh = plsc.VectorSubcoreMesh(
        core_axis_name="c", subcore_axis_name="s",
        num_cores=NC, num_subcores=NS)
    fn = pl.kernel(
        _sc_gather_body,
        out_type=jax.ShapeDtypeStruct((EH, D), jnp.float32),
        mesh=mesh,
        scratch_types=[
            pltpu.VMEM((NBUF, 128), jnp.int32),
            pltpu.VMEM((NBUF, 128), jnp.int32),
            pltpu.VMEM((NBUF, 128), jnp.int32),
            pltpu.VMEM((NBUF * 128, D), jnp.float32),
            pltpu.SemaphoreType.DMA((NBUF,)),
            pltpu.SemaphoreType.DMA((NBUF,)),
            pltpu.SemaphoreType.DMA((NBUF,)),
            pltpu.SemaphoreType.DMA((NBUF,)),
        ],
    )
    return fn(p2, q, src2d, dst2d)


BE = 1280
NB_A = EH // BE
NB_B = (E - EH) // BE


def _silu_heads(g, wcat, b2):
    h = g * (1.0 / (1.0 + jnp.exp(-g)))
    o = jnp.dot(h, wcat, preferred_element_type=jnp.float32) + b2
    return o[:, :D], jnp.exp(o[:, D:])


def _head_body_a(g_ref, wcat_ref, b2_ref, loc_ref, scale_ref):
    loc, scale = _silu_heads(g_ref[...], wcat_ref[...], b2_ref[...])
    loc_ref[...] = loc
    scale_ref[...] = scale


def _head_body_b(g_ref, wcat_ref, b2_ref, li_ref, si_ref, loc_ref, scale_ref):
    del li_ref, si_ref
    loc, scale = _silu_heads(g_ref[...], wcat_ref[...], b2_ref[...])
    loc_ref[...] = loc
    scale_ref[...] = scale


_OUT_SHAPE = [
    jax.ShapeDtypeStruct((E, D), jnp.float32),
    jax.ShapeDtypeStruct((E, D), jnp.float32),
]
_WSPECS = [
    pl.BlockSpec((D, 2 * D), lambda i: (0, 0)),
    pl.BlockSpec((1, 2 * D), lambda i: (0, 0)),
]


def _heads_first(g, wcat, b2, nblocks):
    return pl.pallas_call(
        _head_body_a,
        grid=(nblocks,),
        in_specs=[pl.BlockSpec((BE, D), lambda i: (i, 0))] + _WSPECS,
        out_specs=[
            pl.BlockSpec((BE, D), lambda i: (i, 0)),
            pl.BlockSpec((BE, D), lambda i: (i, 0)),
        ],
        out_shape=_OUT_SHAPE,
        compiler_params=pltpu.CompilerParams(
            dimension_semantics=("arbitrary",)),
    )(g, wcat, b2)


def _heads_next(g, wcat, b2, loc_init, scale_init, off, nblocks):
    return pl.pallas_call(
        _head_body_b,
        grid=(nblocks,),
        in_specs=[pl.BlockSpec((BE, D), lambda i: (i, 0))] + _WSPECS + [
            pl.BlockSpec(memory_space=pl.ANY),
            pl.BlockSpec(memory_space=pl.ANY),
        ],
        out_specs=[
            pl.BlockSpec((BE, D), lambda i, off=off: (i + off, 0)),
            pl.BlockSpec((BE, D), lambda i, off=off: (i + off, 0)),
        ],
        out_shape=_OUT_SHAPE,
        input_output_aliases={3: 0, 4: 1},
        compiler_params=pltpu.CompilerParams(
            dimension_semantics=("arbitrary",)),
    )(g, wcat, b2, loc_init, scale_init)


def kernel(feat, edge_index, W1, b1, W_loc, b_loc, W_ls, b_ls):
    src = edge_index[0].astype(jnp.int32)
    dst = edge_index[1].astype(jnp.int32)
    pad_s = jnp.arange(E4 - E, dtype=jnp.int32)
    src2d = jnp.concatenate([src, pad_s]).reshape(E4 // 128, 128)
    dst2d = jnp.concatenate([dst, pad_s + 1]).reshape(E4 // 128, 128)

    w1s = W1[:D]
    w1d = W1[D:2 * D]
    wfb = jnp.stack([b1, b1 + W1[2 * D]])

    p2, q = _node_projections(feat, w1s, w1d, wfb)

    gs = [_sc_gather(p2, q, src2d[i * NCH:(i + 1) * NCH],
                     dst2d[i * NCH:(i + 1) * NCH]) for i in range(NSLICE)]

    wcat = jnp.concatenate([W_loc, W_ls], axis=1)
    b2 = jnp.concatenate([b_loc, b_ls]).reshape(1, 2 * D)

    blocks_per_slice = EH // BE
    loc, scale = _heads_first(gs[0], wcat, b2, blocks_per_slice)
    for i in range(1, NSLICE):
        off = i * blocks_per_slice
        nb = min(blocks_per_slice, E // BE - off)
        loc, scale = _heads_next(gs[i], wcat, b2, loc, scale, off, nb)
    return (loc, scale)


# speedup vs baseline: 1.0507x; 1.0012x over previous
import jax
import jax.numpy as jnp
from jax import lax
from jax.experimental import pallas as pl
from jax.experimental.pallas import tpu as pltpu
from jax.experimental.pallas import tpu_sc as plsc

N = 10000
E = 160000
D = 128

NC, NS = 2, 16
NW = NC * NS
E4 = 163840
CHUNK = 128
NSLICE = 4
EH = E4 // NSLICE
NCH = EH // CHUNK
TW = NCH // NW
NBUF = 7
GDQ = 2
GDW = 4


def _proj_body(feat_ref, w1s_ref, w1d_ref, wfb_ref, p2_ref, q_ref):
    f = feat_ref[...]
    p = jnp.dot(f, w1s_ref[...], preferred_element_type=jnp.float32)
    p2_ref[0] = p + wfb_ref[0:1, :]
    p2_ref[1] = p + wfb_ref[1:2, :]
    q_ref[...] = jnp.dot(f, w1d_ref[...], preferred_element_type=jnp.float32)


def _node_projections(feat, w1s, w1d, wfb):
    bn = 2000
    grid = (N // bn,)
    p2, q = pl.pallas_call(
        _proj_body,
        grid=grid,
        in_specs=[
            pl.BlockSpec((bn, D), lambda i: (i, 0)),
            pl.BlockSpec((D, D), lambda i: (0, 0)),
            pl.BlockSpec((D, D), lambda i: (0, 0)),
            pl.BlockSpec((2, D), lambda i: (0, 0)),
        ],
        out_specs=[
            pl.BlockSpec((2, bn, D), lambda i: (0, i, 0)),
            pl.BlockSpec((bn, D), lambda i: (i, 0)),
        ],
        out_shape=[
            jax.ShapeDtypeStruct((2, N, D), jnp.float32),
            jax.ShapeDtypeStruct((N, D), jnp.float32),
        ],
    )(feat, w1s, w1d, wfb)
    return p2.reshape(2 * N, D), q


def _sc_gather_body(p2_hbm, q_hbm, src_hbm, dst_hbm, g_hbm,
                    src_v, dst_v, sx_v, buf, semI, semP, semQ, semW):
    cid = lax.axis_index("c")
    sid = lax.axis_index("s")
    ch0 = (cid * NS + sid) * TW
    T = TW

    def fire_idx(c):
        b = lax.rem(c, NBUF)
        pltpu.async_copy(src_hbm.at[ch0 + c], src_v.at[b], semI.at[b])
        pltpu.async_copy(dst_hbm.at[ch0 + c], dst_v.at[b], semI.at[b])

    fire_idx(0)

    def body(t, _):
        b = lax.rem(t, NBUF)

        @pl.when(t < T)
        def _():
            @pl.when(t >= NBUF)
            def _():
                pltpu.make_async_copy(buf.at[pl.ds(b * 128, 128)],
                                      g_hbm.at[pl.ds(0, 128)],
                                      semW.at[b]).wait()
            pltpu.make_async_copy(src_hbm.at[0], src_v.at[b], semI.at[b]).wait()
            pltpu.make_async_copy(dst_hbm.at[0], dst_v.at[b], semI.at[b]).wait()
            for j in range(8):
                s = src_v[b, pl.ds(j * 16, 16)]
                d = dst_v[b, pl.ds(j * 16, 16)]
                sx_v[b, pl.ds(j * 16, 16)] = jnp.where(s == d, s + N, s)
            pltpu.async_copy(p2_hbm.at[sx_v.at[b]],
                             buf.at[pl.ds(b * 128, 128)], semP.at[b])

        cq = t - GDQ

        @pl.when((cq >= 0) & (cq < T))
        def _():
            bq = lax.rem(cq, NBUF)
            pltpu.make_async_copy(g_hbm.at[pl.ds(0, 128)],
                                  buf.at[pl.ds(bq * 128, 128)],
                                  semP.at[bq]).wait()
            pltpu.async_copy(q_hbm.at[dst_v.at[bq]],
                             buf.at[pl.ds(bq * 128, 128)],
                             semQ.at[bq], add=True)

        cw = t - GDW

        @pl.when((cw >= 0) & (cw < T))
        def _():
            bw = lax.rem(cw, NBUF)
            pltpu.make_async_copy(g_hbm.at[pl.ds(0, 128)],
                                  buf.at[pl.ds(bw * 128, 128)],
                                  semQ.at[bw]).wait()
            pltpu.async_copy(buf.at[pl.ds(bw * 128, 128)],
                             g_hbm.at[pl.ds((ch0 + cw) * 128, 128)],
                             semW.at[bw])

        @pl.when(t + 1 < T)
        def _():
            fire_idx(t + 1)

        return 0

    lax.fori_loop(0, T + GDW, body, 0)

    for k in range(min(NBUF, TW)):
        bk = (TW - 1 - k) % NBUF
        pltpu.make_async_copy(buf.at[pl.ds(bk * 128, 128)],
                              g_hbm.at[pl.ds(0, 128)],
                              semW.at[bk]).wait()


def _sc_gather(p2, q, src2d, dst2d):
    mesh = plsc.VectorSubcoreMesh(
        core_axis_name="c", subcore_axis_name="s",
        num_cores=NC, num_subcores=NS)
    fn = pl.kernel(
        _sc_gather_body,
        out_type=jax.ShapeDtypeStruct((EH, D), jnp.float32),
        mesh=mesh,
        scratch_types=[
            pltpu.VMEM((NBUF, 128), jnp.int32),
            pltpu.VMEM((NBUF, 128), jnp.int32),
            pltpu.VMEM((NBUF, 128), jnp.int32),
            pltpu.VMEM((NBUF * 128, D), jnp.float32),
            pltpu.SemaphoreType.DMA((NBUF,)),
            pltpu.SemaphoreType.DMA((NBUF,)),
            pltpu.SemaphoreType.DMA((NBUF,)),
            pltpu.SemaphoreType.DMA((NBUF,)),
        ],
    )
    return fn(p2, q, src2d, dst2d)


BE = 1280
NB_A = EH // BE
NB_B = (E - EH) // BE


def _silu_heads(g, wcat, b2):
    h = g * (1.0 / (1.0 + jnp.exp(-g)))
    o = jnp.dot(h, wcat, preferred_element_type=jnp.float32) + b2
    return o[:, :D], jnp.exp(o[:, D:])


def _head_body_a(g_ref, wcat_ref, b2_ref, loc_ref, scale_ref):
    loc, scale = _silu_heads(g_ref[...], wcat_ref[...], b2_ref[...])
    loc_ref[...] = loc
    scale_ref[...] = scale


def _head_body_b(g_ref, wcat_ref, b2_ref, li_ref, si_ref, loc_ref, scale_ref):
    del li_ref, si_ref
    loc, scale = _silu_heads(g_ref[...], wcat_ref[...], b2_ref[...])
    loc_ref[...] = loc
    scale_ref[...] = scale


_OUT_SHAPE = [
    jax.ShapeDtypeStruct((E, D), jnp.float32),
    jax.ShapeDtypeStruct((E, D), jnp.float32),
]
_WSPECS = [
    pl.BlockSpec((D, 2 * D), lambda i: (0, 0)),
    pl.BlockSpec((1, 2 * D), lambda i: (0, 0)),
]


def _heads_first(g, wcat, b2, nblocks):
    return pl.pallas_call(
        _head_body_a,
        grid=(nblocks,),
        in_specs=[pl.BlockSpec((BE, D), lambda i: (i, 0))] + _WSPECS,
        out_specs=[
            pl.BlockSpec((BE, D), lambda i: (i, 0)),
            pl.BlockSpec((BE, D), lambda i: (i, 0)),
        ],
        out_shape=_OUT_SHAPE,
        compiler_params=pltpu.CompilerParams(
            dimension_semantics=("arbitrary",)),
    )(g, wcat, b2)


def _heads_next(g, wcat, b2, loc_init, scale_init, off, nblocks):
    return pl.pallas_call(
        _head_body_b,
        grid=(nblocks,),
        in_specs=[pl.BlockSpec((BE, D), lambda i: (i, 0))] + _WSPECS + [
            pl.BlockSpec(memory_space=pl.ANY),
            pl.BlockSpec(memory_space=pl.ANY),
        ],
        out_specs=[
            pl.BlockSpec((BE, D), lambda i, off=off: (i + off, 0)),
            pl.BlockSpec((BE, D), lambda i, off=off: (i + off, 0)),
        ],
        out_shape=_OUT_SHAPE,
        input_output_aliases={3: 0, 4: 1},
        compiler_params=pltpu.CompilerParams(
            dimension_semantics=("arbitrary",)),
    )(g, wcat, b2, loc_init, scale_init)


def kernel(feat, edge_index, W1, b1, W_loc, b_loc, W_ls, b_ls):
    src = edge_index[0].astype(jnp.int32)
    dst = edge_index[1].astype(jnp.int32)
    pad_s = jnp.arange(E4 - E, dtype=jnp.int32)
    src2d = jnp.concatenate([src, pad_s]).reshape(E4 // 128, 128)
    dst2d = jnp.concatenate([dst, pad_s + 1]).reshape(E4 // 128, 128)

    w1s = W1[:D]
    w1d = W1[D:2 * D]
    wfb = jnp.stack([b1, b1 + W1[2 * D]])

    p2, q = _node_projections(feat, w1s, w1d, wfb)

    gs = [_sc_gather(p2, q, src2d[i * NCH:(i + 1) * NCH],
                     dst2d[i * NCH:(i + 1) * NCH]) for i in range(NSLICE)]

    wcat = jnp.concatenate([W_loc, W_ls], axis=1)
    b2 = jnp.concatenate([b_loc, b_ls]).reshape(1, 2 * D)

    blocks_per_slice = EH // BE
    loc, scale = _heads_first(gs[0], wcat, b2, blocks_per_slice)
    for i in range(1, NSLICE):
        off = i * blocks_per_slice
        nb = min(blocks_per_slice, E // BE - off)
        loc, scale = _heads_next(gs[i], wcat, b2, loc, scale, off, nb)
    return (loc, scale)
